# Initial kernel scaffold; baseline (speedup 1.0000x reference)
import jax
import jax.numpy as jnp
from jax.experimental import pallas as pl


def kernel(features, segment_ids):
    raise NotImplementedError("write your pallas kernel here")


# speedup vs baseline: 1.4360x; 1.4360x over previous
import functools

import jax
import jax.numpy as jnp
from jax import lax
from jax.experimental import pallas as pl
from jax.experimental.pallas import tpu as pltpu
from jax.experimental.pallas import tpu_sc as plsc

N = 160000
D = 256
NSEG = 64
NC = 2
NS = 16
NW = NC * NS
R = N // NW
CHUNK = 200
NCHUNK = R // CHUNK
NPAIR = (NCHUNK - 1) // 2
LANES = 16
DJ = D // LANES
SEG_PER_TILE = NSEG // NS
L4 = SEG_PER_TILE * D

NEG_INF = float("-inf")


def _tile_body(feat_hbm, ids_hbm, out_hbm,
               ids_v, buf0, buf1, acc_v, sem0, sem1):
    c = lax.axis_index("c")
    s = lax.axis_index("s")
    w = s * NC + c
    base = w * R

    pltpu.sync_copy(ids_hbm.at[pl.ds(base, R)], ids_v.at[pl.ds(0, R)])

    def init_body(i, carry):
        acc_v[pl.ds(i * LANES, LANES)] = jnp.full((LANES,), NEG_INF, jnp.float32)
        return carry
    lax.fori_loop(0, (NSEG * D) // LANES, init_body, 0)

    def process(buf, goff):
        def row_body(r, carry):
            seg = ids_v[pl.ds(goff + r, LANES)][0]
            soff = seg * D
            for j in range(DJ):
                cur = acc_v[pl.ds(soff + j * LANES, LANES)]
                row = buf[r, pl.ds(j * LANES, LANES)]
                acc_v[pl.ds(soff + j * LANES, LANES)] = jnp.maximum(cur, row)
            return carry
        lax.fori_loop(0, CHUNK, row_body, 0)

    pltpu.async_copy(feat_hbm.at[pl.ds(base, CHUNK)], buf0, sem0)

    def chunk_pair(g, carry):
        pltpu.async_copy(
            feat_hbm.at[pl.ds(base + (2 * g + 1) * CHUNK, CHUNK)], buf1, sem1)
        pltpu.make_async_copy(
            feat_hbm.at[pl.ds(base, CHUNK)], buf0, sem0).wait()
        process(buf0, 2 * g * CHUNK)

        pltpu.async_copy(
            feat_hbm.at[pl.ds(base + (2 * g + 2) * CHUNK, CHUNK)], buf0, sem0)
        pltpu.make_async_copy(
            feat_hbm.at[pl.ds(base, CHUNK)], buf1, sem1).wait()
        process(buf1, (2 * g + 1) * CHUNK)
        return carry

    lax.fori_loop(0, NPAIR, chunk_pair, 0)

    pltpu.make_async_copy(feat_hbm.at[pl.ds(base, CHUNK)], buf0, sem0).wait()
    process(buf0, (NCHUNK - 1) * CHUNK)

    pltpu.sync_copy(acc_v, out_hbm.at[w])


@functools.partial(
    pl.kernel,
    out_type=jax.ShapeDtypeStruct((NW, NSEG * D), jnp.float32),
    mesh=plsc.VectorSubcoreMesh(core_axis_name="c", subcore_axis_name="s"),
    scratch_types=[
        pltpu.VMEM((R + LANES,), jnp.int32),
        pltpu.VMEM((CHUNK, D), jnp.float32),
        pltpu.VMEM((CHUNK, D), jnp.float32),
        pltpu.VMEM((NSEG * D,), jnp.float32),
        pltpu.SemaphoreType.DMA,
        pltpu.SemaphoreType.DMA,
    ],
)
def _segmax_sc(feat_hbm, ids_hbm, out_hbm,
               ids_v, buf0, buf1, acc_v, sem0, sem1):
    _tile_body(feat_hbm, ids_hbm, out_hbm,
               ids_v, buf0, buf1, acc_v, sem0, sem1)


def kernel(features, segment_ids):
    ids32 = segment_ids.astype(jnp.int32)
    parts = _segmax_sc(features, ids32)
    return jnp.max(parts.reshape(NW, NSEG, D), axis=0)


# speedup vs baseline: 4.4189x; 3.0773x over previous
import functools

import jax
import jax.numpy as jnp
from jax import lax
from jax.experimental import pallas as pl
from jax.experimental.pallas import tpu as pltpu
from jax.experimental.pallas import tpu_sc as plsc

N = 160000
D = 256
NSEG = 64
NC = 2
NS = 16
NW = NC * NS
R = N // NW
CHUNK = 200
NCHUNK = R // CHUNK
NPAIR = (NCHUNK - 1) // 2
LANES = 16
DJ = D // LANES
G = 8
SEG_PER_TILE = NSEG // NS
L4 = SEG_PER_TILE * D

NEG_INF = float("-inf")


def _tile_body(feat_hbm, ids_hbm, out_hbm,
               ids_v, buf0, buf1, acc_v, cur_v, sem0, sem1):
    c = lax.axis_index("c")
    s = lax.axis_index("s")
    w = s * NC + c
    base = w * R

    pltpu.sync_copy(ids_hbm.at[pl.ds(base, R)], ids_v.at[pl.ds(0, R)])

    def init_body(i, carry):
        acc_v[pl.ds(i * LANES, LANES)] = jnp.full((LANES,), NEG_INF, jnp.float32)
        return carry
    lax.fori_loop(0, (NSEG * D) // LANES, init_body, 0)
    for j in range(DJ):
        cur_v[pl.ds(j * LANES, LANES)] = jnp.full((LANES,), NEG_INF, jnp.float32)

    def flush(cs):
        for j in range(DJ):
            sa = pl.ds(cs * D + j * LANES, LANES)
            sc = pl.ds(j * LANES, LANES)
            acc_v[sa] = jnp.maximum(acc_v[sa], cur_v[sc])
            cur_v[sc] = jnp.full((LANES,), NEG_INF, jnp.float32)

    def process(buf, goff, cs):
        def group_body(k, cs):
            ids16 = ids_v[pl.ds(goff + k * G, LANES)]
            first = ids16[0]
            last = ids16[G - 1]

            def fast(cs):
                @pl.when(first != cs)
                def _():
                    flush(cs)
                for j in range(DJ):
                    sl = pl.ds(j * LANES, LANES)
                    a = cur_v[sl]
                    for r in range(G):
                        a = jnp.maximum(
                            a, buf[k * G + r, pl.ds(j * LANES, LANES)])
                    cur_v[sl] = a
                return first

            def slow(cs):
                for r in range(G):
                    soff = ids16[r] * D
                    for j in range(DJ):
                        sl = pl.ds(soff + j * LANES, LANES)
                        acc_v[sl] = jnp.maximum(
                            acc_v[sl], buf[k * G + r, pl.ds(j * LANES, LANES)])
                return cs

            return lax.cond(first == last, fast, slow, cs)
        return lax.fori_loop(0, CHUNK // G, group_body, cs)

    pltpu.async_copy(feat_hbm.at[pl.ds(base, CHUNK)], buf0, sem0)

    cs = ids_v[pl.ds(0, LANES)][0]

    def chunk_pair(g, cs):
        pltpu.async_copy(
            feat_hbm.at[pl.ds(base + (2 * g + 1) * CHUNK, CHUNK)], buf1, sem1)
        pltpu.make_async_copy(
            feat_hbm.at[pl.ds(base, CHUNK)], buf0, sem0).wait()
        cs = process(buf0, 2 * g * CHUNK, cs)

        pltpu.async_copy(
            feat_hbm.at[pl.ds(base + (2 * g + 2) * CHUNK, CHUNK)], buf0, sem0)
        pltpu.make_async_copy(
            feat_hbm.at[pl.ds(base, CHUNK)], buf1, sem1).wait()
        cs = process(buf1, (2 * g + 1) * CHUNK, cs)
        return cs

    cs = lax.fori_loop(0, NPAIR, chunk_pair, cs)

    pltpu.make_async_copy(feat_hbm.at[pl.ds(base, CHUNK)], buf0, sem0).wait()
    cs = process(buf0, (NCHUNK - 1) * CHUNK, cs)
    flush(cs)

    pltpu.sync_copy(acc_v, out_hbm.at[w])


@functools.partial(
    pl.kernel,
    out_type=jax.ShapeDtypeStruct((NW, NSEG * D), jnp.float32),
    mesh=plsc.VectorSubcoreMesh(core_axis_name="c", subcore_axis_name="s"),
    scratch_types=[
        pltpu.VMEM((R + LANES,), jnp.int32),
        pltpu.VMEM((CHUNK, D), jnp.float32),
        pltpu.VMEM((CHUNK, D), jnp.float32),
        pltpu.VMEM((NSEG * D,), jnp.float32),
        pltpu.VMEM((D,), jnp.float32),
        pltpu.SemaphoreType.DMA,
        pltpu.SemaphoreType.DMA,
    ],
)
def _segmax_sc(feat_hbm, ids_hbm, out_hbm,
               ids_v, buf0, buf1, acc_v, cur_v, sem0, sem1):
    _tile_body(feat_hbm, ids_hbm, out_hbm,
               ids_v, buf0, buf1, acc_v, cur_v, sem0, sem1)


def kernel(features, segment_ids):
    ids32 = segment_ids.astype(jnp.int32)
    parts = _segmax_sc(features, ids32)
    return jnp.max(parts.reshape(NW, NSEG, D), axis=0)


# speedup vs baseline: 5.3043x; 1.2004x over previous
import functools

import jax
import jax.numpy as jnp
from jax import lax
from jax.experimental import pallas as pl
from jax.experimental.pallas import tpu as pltpu
from jax.experimental.pallas import tpu_sc as plsc

N = 160000
D = 256
NSEG = 64
NC = 2
NS = 16
NW = NC * NS
R = N // NW
CHUNK = 200
NCHUNK = R // CHUNK
NPAIR = (NCHUNK - 1) // 2
LANES = 16
DJ = D // LANES
G = 8
SEG_PER_TILE = NSEG // NS
L4 = SEG_PER_TILE * D

NEG_INF = float("-inf")


def _tile_body(feat_hbm, ids_hbm, out_hbm,
               ids_v, buf0, buf1, acc_v, cur_v, sem0, sem1):
    c = lax.axis_index("c")
    s = lax.axis_index("s")
    w = s * NC + c
    base = w * R

    pltpu.sync_copy(ids_hbm.at[pl.ds(base, R)], ids_v.at[pl.ds(0, R)])

    def init_body(i, carry):
        acc_v[pl.ds(i * LANES, LANES)] = jnp.full((LANES,), NEG_INF, jnp.float32)
        return carry
    lax.fori_loop(0, (NSEG * D) // LANES, init_body, 0)
    for j in range(DJ):
        cur_v[pl.ds(j * LANES, LANES)] = jnp.full((LANES,), NEG_INF, jnp.float32)

    def flush(cs):
        for j in range(DJ):
            sa = pl.ds(cs * D + j * LANES, LANES)
            sc = pl.ds(j * LANES, LANES)
            acc_v[sa] = jnp.maximum(acc_v[sa], cur_v[sc])
            cur_v[sc] = jnp.full((LANES,), NEG_INF, jnp.float32)

    def process(buf, goff, cs):
        cfirst = ids_v[pl.ds(goff, LANES)][0]
        clast = ids_v[pl.ds(goff + CHUNK - 1, LANES)][0]

        def fast_chunk(cs):
            @pl.when(cfirst != cs)
            def _():
                flush(cs)

            def grp(k, accs):
                out = []
                for j in range(DJ):
                    a = accs[j]
                    for r in range(G):
                        a = jnp.maximum(
                            a, buf[k * G + r, pl.ds(j * LANES, LANES)])
                    out.append(a)
                return tuple(out)

            accs = tuple(cur_v[pl.ds(j * LANES, LANES)] for j in range(DJ))
            accs = lax.fori_loop(0, CHUNK // G, grp, accs)
            for j in range(DJ):
                cur_v[pl.ds(j * LANES, LANES)] = accs[j]
            return cfirst

        def slow_chunk(cs):
            def group_body(k, cs):
                ids16 = ids_v[pl.ds(goff + k * G, LANES)]
                first = ids16[0]
                last = ids16[G - 1]

                def fast(cs):
                    @pl.when(first != cs)
                    def _():
                        flush(cs)
                    for j in range(DJ):
                        sl = pl.ds(j * LANES, LANES)
                        a = cur_v[sl]
                        for r in range(G):
                            a = jnp.maximum(
                                a, buf[k * G + r, pl.ds(j * LANES, LANES)])
                        cur_v[sl] = a
                    return first

                def slow(cs):
                    for r in range(G):
                        soff = ids16[r] * D
                        for j in range(DJ):
                            sl = pl.ds(soff + j * LANES, LANES)
                            acc_v[sl] = jnp.maximum(
                                acc_v[sl],
                                buf[k * G + r, pl.ds(j * LANES, LANES)])
                    return cs

                return lax.cond(first == last, fast, slow, cs)
            return lax.fori_loop(0, CHUNK // G, group_body, cs)

        return lax.cond(cfirst == clast, fast_chunk, slow_chunk, cs)

    pltpu.async_copy(feat_hbm.at[pl.ds(base, CHUNK)], buf0, sem0)

    cs = ids_v[pl.ds(0, LANES)][0]

    def chunk_pair(g, cs):
        pltpu.async_copy(
            feat_hbm.at[pl.ds(base + (2 * g + 1) * CHUNK, CHUNK)], buf1, sem1)
        pltpu.make_async_copy(
            feat_hbm.at[pl.ds(base, CHUNK)], buf0, sem0).wait()
        cs = process(buf0, 2 * g * CHUNK, cs)

        pltpu.async_copy(
            feat_hbm.at[pl.ds(base + (2 * g + 2) * CHUNK, CHUNK)], buf0, sem0)
        pltpu.make_async_copy(
            feat_hbm.at[pl.ds(base, CHUNK)], buf1, sem1).wait()
        cs = process(buf1, (2 * g + 1) * CHUNK, cs)
        return cs

    cs = lax.fori_loop(0, NPAIR, chunk_pair, cs)

    pltpu.make_async_copy(feat_hbm.at[pl.ds(base, CHUNK)], buf0, sem0).wait()
    cs = process(buf0, (NCHUNK - 1) * CHUNK, cs)
    flush(cs)

    pltpu.sync_copy(acc_v, out_hbm.at[w])


@functools.partial(
    pl.kernel,
    out_type=jax.ShapeDtypeStruct((NW, NSEG * D), jnp.float32),
    mesh=plsc.VectorSubcoreMesh(core_axis_name="c", subcore_axis_name="s"),
    scratch_types=[
        pltpu.VMEM((R + LANES,), jnp.int32),
        pltpu.VMEM((CHUNK, D), jnp.float32),
        pltpu.VMEM((CHUNK, D), jnp.float32),
        pltpu.VMEM((NSEG * D,), jnp.float32),
        pltpu.VMEM((D,), jnp.float32),
        pltpu.SemaphoreType.DMA,
        pltpu.SemaphoreType.DMA,
    ],
)
def _segmax_sc(feat_hbm, ids_hbm, out_hbm,
               ids_v, buf0, buf1, acc_v, cur_v, sem0, sem1):
    _tile_body(feat_hbm, ids_hbm, out_hbm,
               ids_v, buf0, buf1, acc_v, cur_v, sem0, sem1)


def kernel(features, segment_ids):
    ids32 = segment_ids.astype(jnp.int32)
    parts = _segmax_sc(features, ids32)
    return jnp.max(parts.reshape(NW, NSEG, D), axis=0)
